# Initial kernel scaffold; baseline (speedup 1.0000x reference)
import jax
import jax.numpy as jnp
from jax.experimental import pallas as pl


def kernel(alpha, mu, D, subj_id):
    raise NotImplementedError("write your pallas kernel here")


# speedup vs baseline: 1.6484x; 1.6484x over previous
import functools

import jax
import jax.numpy as jnp
from jax import lax
from jax.experimental import pallas as pl
from jax.experimental.pallas import tpu as pltpu
from jax.experimental.pallas import tpu_sc as plsc

_LANES = 16


def _full16(val):
    return jnp.full((_LANES,), val, dtype=jnp.int32)


def kernel(alpha, mu, D, subj_id):
    B, T, M = alpha.shape
    S, _, C = mu.shape
    P = B * T
    ROWS = S * M
    CC = C * C

    info = plsc.get_sparse_core_info()
    NC, NS = info.num_cores, info.num_subcores
    NW = NC * NS
    PPW = P // NW

    af = alpha.reshape(P, M)
    sid = subj_id.reshape(P).astype(jnp.int32)
    ridx = sid[:, None] * M + jnp.arange(M, dtype=jnp.int32)[None, :]
    mur = mu.reshape(ROWS, C)
    Dr = D.reshape(ROWS, CC)

    mesh = plsc.VectorSubcoreMesh(core_axis_name="c", subcore_axis_name="s")

    @functools.partial(
        pl.kernel,
        out_type=(
            jax.ShapeDtypeStruct((P, C), jnp.float32),
            jax.ShapeDtypeStruct((P, CC), jnp.float32),
        ),
        mesh=mesh,
        compiler_params=pltpu.CompilerParams(use_tc_tiling_on_sc=False,
                                          needs_layout_passes=False),
        scratch_types=[
            pltpu.VMEM((PPW, M), jnp.int32),
            pltpu.VMEM((PPW, M), jnp.float32),
            pltpu.VMEM((ROWS, C), jnp.float32),
            pltpu.VMEM((M, CC), jnp.float32),
            pltpu.VMEM((M, CC), jnp.float32),
            pltpu.VMEM((CC,), jnp.float32),
            pltpu.VMEM((PPW, C), jnp.float32),
            pltpu.SemaphoreType.DMA,
            pltpu.SemaphoreType.DMA,
        ],
    )
    def sc_combine(af_hbm, ridx_hbm, mur_hbm, dr_hbm, m_hbm, cov_hbm,
                   idx_v, alpha_v, mur_v, dbuf_a, dbuf_b, cov_buf, mout_v,
                   sem_a, sem_b):
        wid = lax.axis_index("s") * NC + lax.axis_index("c")
        base = wid * PPW

        pltpu.sync_copy(ridx_hbm.at[pl.ds(base, PPW)], idx_v)
        pltpu.sync_copy(af_hbm.at[pl.ds(base, PPW)], alpha_v)
        pltpu.sync_copy(mur_hbm, mur_v)

        pltpu.async_copy(dr_hbm.at[idx_v.at[0]], dbuf_a, sem_a)
        pltpu.async_copy(dr_hbm.at[idx_v.at[1]], dbuf_b, sem_b)

        col_iota = lax.iota(jnp.int32, _LANES)

        def compute_pair(i, dbuf):
            av = [plsc.load_gather(alpha_v, [_full16(i), _full16(m)])
                  for m in range(M)]

            def chunk_body(j, carry):
                col = j * _LANES
                acc = dbuf[0, pl.ds(col, _LANES)] * av[0]
                for m in range(1, M):
                    acc = acc + dbuf[m, pl.ds(col, _LANES)] * av[m]
                cov_buf[pl.ds(col, _LANES)] = acc
                return carry

            lax.fori_loop(0, CC // _LANES, chunk_body, 0, unroll=2)

            rows = [plsc.load_gather(idx_v, [_full16(i), _full16(m)])
                    for m in range(M)]
            for c in range(C // _LANES):
                cols = col_iota + (c * _LANES)
                acc = plsc.load_gather(mur_v, [rows[0], cols]) * av[0]
                for m in range(1, M):
                    acc = acc + plsc.load_gather(mur_v, [rows[m], cols]) * av[m]
                mout_v[i, pl.ds(c * _LANES, _LANES)] = acc

        def body(g, carry):
            i0 = 2 * g
            i1 = 2 * g + 1
            nxt0 = jnp.minimum(i0 + 2, PPW - 1)
            nxt1 = jnp.minimum(i1 + 2, PPW - 1)

            pltpu.make_async_copy(dr_hbm.at[idx_v.at[i0]], dbuf_a, sem_a).wait()
            compute_pair(i0, dbuf_a)
            pltpu.sync_copy(cov_buf, cov_hbm.at[base + i0])
            pltpu.async_copy(dr_hbm.at[idx_v.at[nxt0]], dbuf_a, sem_a)

            pltpu.make_async_copy(dr_hbm.at[idx_v.at[i1]], dbuf_b, sem_b).wait()
            compute_pair(i1, dbuf_b)
            pltpu.sync_copy(cov_buf, cov_hbm.at[base + i1])
            pltpu.async_copy(dr_hbm.at[idx_v.at[nxt1]], dbuf_b, sem_b)
            return carry

        lax.fori_loop(0, PPW // 2, body, 0)

        pltpu.make_async_copy(dr_hbm.at[idx_v.at[0]], dbuf_a, sem_a).wait()
        pltpu.make_async_copy(dr_hbm.at[idx_v.at[0]], dbuf_b, sem_b).wait()

        pltpu.sync_copy(mout_v, m_hbm.at[pl.ds(base, PPW)])

    m2d, cov2d = sc_combine(af, ridx, mur, Dr)
    return m2d.reshape(B, T, C), cov2d.reshape(B, T, C, C)
